# baseline (speedup 1.0000x reference)
import functools

import jax
import jax.numpy as jnp
from jax import lax
from jax.experimental import pallas as pl
from jax.experimental.pallas import tpu as pltpu
from jax.experimental.pallas import tpu_sc as plsc

_START = 101
_END = 102
_SEP = 102
_HALF = 2048
_LANES = 16
_CHUNKS = _HALF // _LANES

_MESH = plsc.VectorSubcoreMesh(core_axis_name="c", subcore_axis_name="s")


@functools.partial(
    pl.kernel,
    mesh=_MESH,
    out_type=[
        jax.ShapeDtypeStruct((16, 2 * _HALF), jnp.int32),
        jax.ShapeDtypeStruct((16, 2 * _HALF), jnp.int32),
    ],
    scratch_types=[
        pltpu.VMEM((_HALF + 128,), jnp.int32),
        pltpu.VMEM((_HALF,), jnp.int32),
        pltpu.VMEM((_HALF,), jnp.int32),
        pltpu.SemaphoreType.DMA,
        pltpu.SemaphoreType.DMA,
    ],
    compiler_params=pltpu.CompilerParams(
        needs_layout_passes=False, skip_device_barrier=True
    ),
)
def _pack_sc(seg1, seg2, tok_out, sid_out, src_v, tok_v, sid_v, sem_in, sem_sid):
    half = lax.axis_index("c")
    row = lax.axis_index("s")
    col0 = half * _HALF

    src_body = src_v.at[pl.ds(128, _HALF)]

    @pl.when(half == 0)
    def _():
        pltpu.async_copy(seg1.at[row], src_body, sem_in)

    @pl.when(half == 1)
    def _():
        pltpu.async_copy(seg2.at[row], src_body, sem_in)

    lane = lax.iota(jnp.int32, _LANES)
    first_val = jnp.where(half == 0, jnp.int32(_START), jnp.int32(_SEP))
    is_second = (half == 1).astype(jnp.int32)

    sid_fill = jnp.broadcast_to(is_second, (_LANES,))

    @plsc.parallel_loop(0, _CHUNKS, unroll=8)
    def _(j):
        sid_v[pl.ds(j * _LANES, _LANES)] = sid_fill

    sid_v[pl.ds(0, _LANES)] = jnp.where(lane == 0, jnp.int32(0), is_second)

    sid_cp = pltpu.async_copy(sid_v, sid_out.at[row, pl.ds(col0, _HALF)], sem_sid)

    pltpu.make_async_copy(seg1.at[row], src_body, sem_in).wait()

    @plsc.parallel_loop(0, _CHUNKS, unroll=8)
    def _(j):
        v = src_v[pl.ds(j * _LANES + 127, _LANES)]
        tok_v[pl.ds(j * _LANES, _LANES)] = v

    v0 = tok_v[pl.ds(0, _LANES)]
    tok_v[pl.ds(0, _LANES)] = jnp.where(lane == 0, first_val, v0)
    tail0 = _HALF - _LANES
    vt = tok_v[pl.ds(tail0, _LANES)]
    fix_end = (lane == _LANES - 1) & (half == 1)
    tok_v[pl.ds(tail0, _LANES)] = jnp.where(fix_end, jnp.int32(_END), vt)

    pltpu.sync_copy(tok_v, tok_out.at[row, pl.ds(col0, _HALF)])
    sid_cp.wait()


def kernel(seg1, seg2):
    tokens, segment_ids = _pack_sc(seg1, seg2)
    return tokens, segment_ids


# speedup vs baseline: 1.0363x; 1.0363x over previous
import functools

import jax
import jax.numpy as jnp
from jax import lax
from jax.experimental import pallas as pl
from jax.experimental.pallas import tpu as pltpu
from jax.experimental.pallas import tpu_sc as plsc

_START = 101
_END = 102
_SEP = 102
_HALF = 2048
_LANES = 16
_CHUNKS = _HALF // _LANES

_MESH = plsc.VectorSubcoreMesh(
    core_axis_name="c", subcore_axis_name="s", num_cores=1
)


@functools.partial(
    pl.kernel,
    mesh=_MESH,
    out_type=[
        jax.ShapeDtypeStruct((16, 2 * _HALF), jnp.int32),
        jax.ShapeDtypeStruct((16, 2 * _HALF), jnp.int32),
    ],
    scratch_types=[
        pltpu.VMEM((_HALF + 128,), jnp.int32),
        pltpu.VMEM((_HALF + 128,), jnp.int32),
        pltpu.VMEM((_HALF,), jnp.int32),
        pltpu.VMEM((_HALF,), jnp.int32),
        pltpu.SemaphoreType.DMA,
        pltpu.SemaphoreType.DMA,
    ],
    compiler_params=pltpu.CompilerParams(
        needs_layout_passes=False, skip_device_barrier=True
    ),
)
def _pack_sc(seg1, seg2, tok_out, sid_out, src0_v, src1_v, tok_v, sid_v, sem_in, sem_sid):
    row = lax.axis_index("s")
    lane = lax.iota(jnp.int32, _LANES)

    pltpu.async_copy(seg1.at[row], src0_v.at[pl.ds(128, _HALF)], sem_in)
    pltpu.async_copy(seg2.at[row], src1_v.at[pl.ds(128, _HALF)], sem_in)

    zero_fill = jnp.broadcast_to(jnp.int32(0), (_LANES,))
    one_fill = jnp.broadcast_to(jnp.int32(1), (_LANES,))

    @plsc.parallel_loop(0, _CHUNKS, unroll=8)
    def _(j):
        sid_v[pl.ds(j * _LANES, _LANES)] = zero_fill

    sid_cp0 = pltpu.async_copy(sid_v, sid_out.at[row, pl.ds(0, _HALF)], sem_sid)

    pltpu.make_async_copy(seg1.at[row], src0_v.at[pl.ds(128, _HALF)], sem_in).wait()
    pltpu.make_async_copy(seg2.at[row], src1_v.at[pl.ds(128, _HALF)], sem_in).wait()

    for half, src_v in ((0, src0_v), (1, src1_v)):
        col0 = half * _HALF
        first_val = jnp.int32(_START if half == 0 else _SEP)

        @plsc.parallel_loop(0, _CHUNKS, unroll=8)
        def _(j, _src=src_v):
            v = _src[pl.ds(j * _LANES + 127, _LANES)]
            tok_v[pl.ds(j * _LANES, _LANES)] = v

        v0 = tok_v[pl.ds(0, _LANES)]
        tok_v[pl.ds(0, _LANES)] = jnp.where(lane == 0, first_val, v0)
        if half == 1:
            tail0 = _HALF - _LANES
            vt = tok_v[pl.ds(tail0, _LANES)]
            tok_v[pl.ds(tail0, _LANES)] = jnp.where(
                lane == _LANES - 1, jnp.int32(_END), vt
            )
        pltpu.sync_copy(tok_v, tok_out.at[row, pl.ds(col0, _HALF)])

    sid_cp0.wait()

    @plsc.parallel_loop(0, _CHUNKS, unroll=8)
    def _(j):
        sid_v[pl.ds(j * _LANES, _LANES)] = one_fill

    sid_v[pl.ds(0, _LANES)] = jnp.where(lane == 0, jnp.int32(0), jnp.int32(1))
    pltpu.sync_copy(sid_v, sid_out.at[row, pl.ds(_HALF, _HALF)])


def kernel(seg1, seg2):
    tokens, segment_ids = _pack_sc(seg1, seg2)
    return tokens, segment_ids


# speedup vs baseline: 1.0598x; 1.0226x over previous
import functools

import jax
import jax.numpy as jnp
from jax import lax
from jax.experimental import pallas as pl
from jax.experimental.pallas import tpu as pltpu
from jax.experimental.pallas import tpu_sc as plsc

_START = 101
_END = 102
_SEP = 102
_HALF = 2048
_LANES = 16
_CHUNKS = _HALF // _LANES
_PAD = 128

_MESH = plsc.VectorSubcoreMesh(
    core_axis_name="c", subcore_axis_name="s", num_cores=1
)


@functools.partial(
    pl.kernel,
    mesh=_MESH,
    out_type=[
        jax.ShapeDtypeStruct((16, 2 * _HALF), jnp.int32),
        jax.ShapeDtypeStruct((16, 2 * _HALF), jnp.int32),
    ],
    scratch_types=[
        pltpu.VMEM((_HALF + _PAD,), jnp.int32),
        pltpu.VMEM((_HALF + _PAD,), jnp.int32),
        pltpu.VMEM((_HALF,), jnp.int32),
        pltpu.VMEM((_HALF,), jnp.int32),
        pltpu.VMEM((_HALF,), jnp.int32),
        pltpu.VMEM((_HALF,), jnp.int32),
        pltpu.SemaphoreType.DMA,
        pltpu.SemaphoreType.DMA,
    ],
    compiler_params=pltpu.CompilerParams(
        needs_layout_passes=False, skip_device_barrier=True
    ),
)
def _pack_sc(
    seg1, seg2, tok_out, sid_out,
    src0_v, src1_v, tok0_v, tok1_v, sid0_v, sid1_v, sem_in, sem_out,
):
    row = lax.axis_index("s")
    lane = lax.iota(jnp.int32, _LANES)

    in0 = pltpu.async_copy(seg1.at[row], src0_v.at[pl.ds(_PAD, _HALF)], sem_in)
    in1 = pltpu.async_copy(seg2.at[row], src1_v.at[pl.ds(_PAD, _HALF)], sem_in)

    zero_fill = jnp.broadcast_to(jnp.int32(0), (_LANES,))
    one_fill = jnp.broadcast_to(jnp.int32(1), (_LANES,))

    @plsc.parallel_loop(0, _CHUNKS, unroll=8)
    def _(j):
        sid0_v[pl.ds(j * _LANES, _LANES)] = zero_fill

    cp_sid0 = pltpu.async_copy(sid0_v, sid_out.at[row, pl.ds(0, _HALF)], sem_out)

    @plsc.parallel_loop(0, _CHUNKS, unroll=8)
    def _(j):
        sid1_v[pl.ds(j * _LANES, _LANES)] = one_fill

    sid1_v[pl.ds(0, _LANES)] = jnp.where(lane == 0, jnp.int32(0), jnp.int32(1))
    cp_sid1 = pltpu.async_copy(sid1_v, sid_out.at[row, pl.ds(_HALF, _HALF)], sem_out)

    in0.wait()

    @plsc.parallel_loop(0, _CHUNKS, unroll=8)
    def _(j):
        tok0_v[pl.ds(j * _LANES, _LANES)] = src0_v[pl.ds(j * _LANES + _PAD - 1, _LANES)]

    v0 = tok0_v[pl.ds(0, _LANES)]
    tok0_v[pl.ds(0, _LANES)] = jnp.where(lane == 0, jnp.int32(_START), v0)
    cp_tok0 = pltpu.async_copy(tok0_v, tok_out.at[row, pl.ds(0, _HALF)], sem_out)

    in1.wait()

    @plsc.parallel_loop(0, _CHUNKS, unroll=8)
    def _(j):
        tok1_v[pl.ds(j * _LANES, _LANES)] = src1_v[pl.ds(j * _LANES + _PAD - 1, _LANES)]

    v1 = tok1_v[pl.ds(0, _LANES)]
    tok1_v[pl.ds(0, _LANES)] = jnp.where(lane == 0, jnp.int32(_SEP), v1)
    tail0 = _HALF - _LANES
    vt = tok1_v[pl.ds(tail0, _LANES)]
    tok1_v[pl.ds(tail0, _LANES)] = jnp.where(lane == _LANES - 1, jnp.int32(_END), vt)
    cp_tok1 = pltpu.async_copy(tok1_v, tok_out.at[row, pl.ds(_HALF, _HALF)], sem_out)

    cp_sid0.wait()
    cp_sid1.wait()
    cp_tok0.wait()
    cp_tok1.wait()


def kernel(seg1, seg2):
    tokens, segment_ids = _pack_sc(seg1, seg2)
    return tokens, segment_ids
